# baseline (speedup 1.0000x reference)
import numpy as np
import jax
import jax.numpy as jnp
from jax.experimental import pallas as pl
from jax.experimental.pallas import tpu as pltpu

N = 10000
D_IN = 128
D_HID = 64
D_OUT = 40
P_DROP = 0.5
ROWS = 200
NB = N // ROWS
CB = 5
UNC = NB - CB
SLOTS = 2


def _make_scale():
    keep = jax.random.bernoulli(jax.random.key(42), 1.0 - P_DROP, (N, D_HID))
    return jnp.where(keep, 1.0 / (1.0 - P_DROP), 0.0).astype(jnp.bfloat16)


try:
    with jax.default_device(jax.local_devices(backend="cpu")[0]):
        _SCALE = np.asarray(jax.jit(_make_scale)())
except Exception:
    _SCALE = np.asarray(_make_scale())


_FETCHES = [(b, b % SLOTS) for b in range(NB)] \
    + [(b, (NB + b) % SLOTS) for b in range(UNC)]


def _body(adj_hbm, x_ref, w1_ref, b1_ref, scale_ref, w2_ref, b2_ref,
          out_ref, buf, cache, s1, mid, s2b, bsem):
    def copy(b, slot):
        return pltpu.make_async_copy(
            adj_hbm.at[pl.ds(b * ROWS, ROWS), :], buf.at[slot],
            bsem.at[slot])

    copy(*_FETCHES[0]).start()
    copy(*_FETCHES[1]).start()
    s1[:] = jnp.dot(x_ref[:], w1_ref[:], preferred_element_type=jnp.float32)

    for t in range(NB):
        b, slot = _FETCHES[t]
        copy(b, slot).wait()
        blk = buf[slot]
        m = jnp.dot(blk, s1[:], preferred_element_type=jnp.float32)
        sc = scale_ref[b * ROWS:(b + 1) * ROWS, :].astype(jnp.float32)
        mid[b * ROWS:(b + 1) * ROWS, :] = jnp.maximum(
            (m + b1_ref[:]) * sc, 0.0)
        if b >= UNC:
            cache[b - UNC] = blk.astype(jnp.bfloat16)
        if t + 2 < len(_FETCHES):
            copy(*_FETCHES[t + 2]).start()

    s2b[:] = jnp.dot(mid[:], w2_ref[:],
                     preferred_element_type=jnp.float32).astype(jnp.bfloat16)

    for b in range(NB):
        if b < UNC:
            slot = _FETCHES[NB + b][1]
            copy(b, slot).wait()
            o = jnp.dot(buf[slot], s2b[:].astype(jnp.float32),
                        preferred_element_type=jnp.float32)
            t = NB + b
            if t + 2 < len(_FETCHES):
                copy(*_FETCHES[t + 2]).start()
        else:
            o = jnp.dot(cache[b - UNC], s2b[:],
                        preferred_element_type=jnp.float32)
        o = o + b2_ref[:]
        o = o - jnp.max(o, axis=1, keepdims=True)
        out_ref[b * ROWS:(b + 1) * ROWS, :] = (
            o - jnp.log(jnp.sum(jnp.exp(o), axis=1, keepdims=True)))


def kernel(input, adj, W1, b1, W2, b2):
    x = input.astype(jnp.float32)
    scale = jnp.asarray(_SCALE)

    out = pl.pallas_call(
        _body,
        in_specs=[
            pl.BlockSpec(memory_space=pl.ANY),
            pl.BlockSpec((N, D_IN), lambda: (0, 0)),
            pl.BlockSpec((D_IN, D_HID), lambda: (0, 0)),
            pl.BlockSpec((1, D_HID), lambda: (0, 0)),
            pl.BlockSpec((N, D_HID), lambda: (0, 0)),
            pl.BlockSpec((D_HID, D_OUT), lambda: (0, 0)),
            pl.BlockSpec((1, D_OUT), lambda: (0, 0)),
        ],
        out_specs=pl.BlockSpec((N, D_OUT), lambda: (0, 0)),
        out_shape=jax.ShapeDtypeStruct((N, D_OUT), jnp.float32),
        scratch_shapes=[
            pltpu.VMEM((SLOTS, ROWS, N), jnp.float32),
            pltpu.VMEM((CB, ROWS, N), jnp.bfloat16),
            pltpu.VMEM((N, D_HID), jnp.float32),
            pltpu.VMEM((N, D_HID), jnp.float32),
            pltpu.VMEM((N, D_OUT), jnp.bfloat16),
            pltpu.SemaphoreType.DMA((SLOTS,)),
        ],
        compiler_params=pltpu.CompilerParams(
            vmem_limit_bytes=63 * 1024 * 1024,
        ),
    )(adj, x, W1, b1.reshape(1, D_HID), scale, W2, b2.reshape(1, D_OUT))
    return out


# speedup vs baseline: 1.1065x; 1.1065x over previous
import numpy as np
import jax
import jax.numpy as jnp
from jax.experimental import pallas as pl
from jax.experimental.pallas import tpu as pltpu

N = 10000
D_IN = 128
D_HID = 64
D_OUT = 40
P_DROP = 0.5
ROWS = 200
HALF = N // ROWS
CB = 4
UNC = HALF - CB


def _make_scale():
    keep = jax.random.bernoulli(jax.random.key(42), 1.0 - P_DROP, (N, D_HID))
    return jnp.where(keep, 1.0 / (1.0 - P_DROP), 0.0).astype(jnp.bfloat16)


try:
    with jax.default_device(jax.local_devices(backend="cpu")[0]):
        _SCALE = np.asarray(jax.jit(_make_scale)())
except Exception:
    _SCALE = np.asarray(_make_scale())


def _adj_index(i):
    j = i - HALF
    return (jnp.where(i < HALF, i, jnp.minimum(j, UNC - 1)), 0)


def _body(adj_ref, x_hbm, w1_ref, b1_ref, scale_ref, w2_ref, b2_ref,
          out_ref, x_vmem, s1, mid, s2f, s2b, cache, sem):
    i = pl.program_id(0)

    @pl.when(i == 0)
    def _():
        cp = pltpu.make_async_copy(x_hbm, x_vmem, sem)
        cp.start()
        cp.wait()
        s1[:] = jnp.dot(x_vmem[:], w1_ref[:],
                        preferred_element_type=jnp.float32)

    @pl.when(i < HALF)
    def _():
        blk = adj_ref[:]
        m = jnp.dot(blk, s1[:], preferred_element_type=jnp.float32)
        mid[pl.ds(i * ROWS, ROWS), :] = jnp.maximum(
            (m + b1_ref[:]) * scale_ref[:].astype(jnp.float32), 0.0)

        @pl.when(i >= UNC)
        def _():
            cache[i - UNC] = blk.astype(jnp.bfloat16)

    @pl.when(i == HALF)
    def _():
        s2 = jnp.dot(mid[:], w2_ref[:], preferred_element_type=jnp.float32)
        s2f[:] = s2
        s2b[:] = s2.astype(jnp.bfloat16)

    def _finish(o):
        o = o + b2_ref[:]
        o = o - jnp.max(o, axis=1, keepdims=True)
        out_ref[:] = o - jnp.log(jnp.sum(jnp.exp(o), axis=1, keepdims=True))

    @pl.when((i >= HALF) & (i < HALF + UNC))
    def _():
        _finish(jnp.dot(adj_ref[:], s2f[:],
                        preferred_element_type=jnp.float32))

    @pl.when(i >= HALF + UNC)
    def _():
        _finish(jnp.dot(cache[i - HALF - UNC], s2b[:],
                        preferred_element_type=jnp.float32))


def kernel(input, adj, W1, b1, W2, b2):
    x = input.astype(jnp.float32)
    scale = jnp.asarray(_SCALE)

    out = pl.pallas_call(
        _body,
        grid=(2 * HALF,),
        in_specs=[
            pl.BlockSpec((ROWS, N), _adj_index),
            pl.BlockSpec(memory_space=pl.ANY),
            pl.BlockSpec((D_IN, D_HID), lambda i: (0, 0)),
            pl.BlockSpec((1, D_HID), lambda i: (0, 0)),
            pl.BlockSpec((ROWS, D_HID),
                         lambda i: (jnp.where(i < HALF, i, HALF - 1), 0)),
            pl.BlockSpec((D_HID, D_OUT), lambda i: (0, 0)),
            pl.BlockSpec((1, D_OUT), lambda i: (0, 0)),
        ],
        out_specs=pl.BlockSpec(
            (ROWS, D_OUT), lambda i: (jnp.where(i < HALF, 0, i - HALF), 0)),
        out_shape=jax.ShapeDtypeStruct((N, D_OUT), jnp.float32),
        scratch_shapes=[
            pltpu.VMEM((N, D_IN), jnp.float32),
            pltpu.VMEM((N, D_HID), jnp.float32),
            pltpu.VMEM((N, D_HID), jnp.float32),
            pltpu.VMEM((N, D_OUT), jnp.float32),
            pltpu.VMEM((N, D_OUT), jnp.bfloat16),
            pltpu.VMEM((CB, ROWS, N), jnp.bfloat16),
            pltpu.SemaphoreType.DMA,
        ],
        compiler_params=pltpu.CompilerParams(
            dimension_semantics=("arbitrary",),
            vmem_limit_bytes=63 * 1024 * 1024,
        ),
    )(adj, x, W1, b1.reshape(1, D_HID), scale, W2, b2.reshape(1, D_OUT))
    return out


# speedup vs baseline: 1.1254x; 1.0171x over previous
import numpy as np
import jax
import jax.numpy as jnp
from jax.experimental import pallas as pl
from jax.experimental.pallas import tpu as pltpu

N = 10000
D_IN = 128
D_HID = 64
D_OUT = 40
P_DROP = 0.5
ROWS = 400
HALF = N // ROWS
CB = 1
UNC = HALF - CB


def _make_scale():
    keep = jax.random.bernoulli(jax.random.key(42), 1.0 - P_DROP, (N, D_HID))
    return jnp.where(keep, 1.0 / (1.0 - P_DROP), 0.0).astype(jnp.bfloat16)


try:
    with jax.default_device(jax.local_devices(backend="cpu")[0]):
        _SCALE = np.asarray(jax.jit(_make_scale)())
except Exception:
    _SCALE = np.asarray(_make_scale())


def _adj_index(i):
    j = i - HALF
    return (jnp.where(i < HALF, i, jnp.minimum(j, UNC - 1)), 0)


def _body(adj_ref, x_hbm, w1_ref, b1_ref, scale_ref, w2_ref, b2_ref,
          out_ref, x_vmem, s1, mid, s2b, cache, sem):
    i = pl.program_id(0)

    @pl.when(i == 0)
    def _():
        cp = pltpu.make_async_copy(x_hbm, x_vmem, sem)
        cp.start()
        cp.wait()
        s1[:] = jnp.dot(x_vmem[:], w1_ref[:],
                        preferred_element_type=jnp.float32)

    @pl.when(i < HALF)
    def _():
        m = jnp.dot(adj_ref[:], s1[:], preferred_element_type=jnp.float32)
        mid[pl.ds(i * ROWS, ROWS), :] = jnp.maximum(
            (m + b1_ref[:]) * scale_ref[:].astype(jnp.float32), 0.0)

        @pl.when(i >= UNC)
        def _():
            cache[i - UNC] = adj_ref[:].astype(jnp.bfloat16)

    @pl.when(i == HALF)
    def _():
        s2 = jnp.dot(mid[:], w2_ref[:], preferred_element_type=jnp.float32)
        mid[:, :D_OUT] = s2
        s2b[:] = s2.astype(jnp.bfloat16)

    def _finish(o):
        o = o + b2_ref[:]
        o = o - jnp.max(o, axis=1, keepdims=True)
        out_ref[:] = o - jnp.log(jnp.sum(jnp.exp(o), axis=1, keepdims=True))

    @pl.when((i >= HALF) & (i < HALF + UNC))
    def _():
        _finish(jnp.dot(adj_ref[:], mid[:, :D_OUT],
                        preferred_element_type=jnp.float32))

    @pl.when(i >= HALF + UNC)
    def _():
        _finish(jnp.dot(cache[i - HALF - UNC], s2b[:],
                        preferred_element_type=jnp.float32))


def kernel(input, adj, W1, b1, W2, b2):
    x = input.astype(jnp.float32)
    scale = jnp.asarray(_SCALE)

    out = pl.pallas_call(
        _body,
        grid=(2 * HALF,),
        in_specs=[
            pl.BlockSpec((ROWS, N), _adj_index),
            pl.BlockSpec(memory_space=pl.ANY),
            pl.BlockSpec((D_IN, D_HID), lambda i: (0, 0)),
            pl.BlockSpec((1, D_HID), lambda i: (0, 0)),
            pl.BlockSpec((ROWS, D_HID),
                         lambda i: (jnp.where(i < HALF, i, HALF - 1), 0)),
            pl.BlockSpec((D_HID, D_OUT), lambda i: (0, 0)),
            pl.BlockSpec((1, D_OUT), lambda i: (0, 0)),
        ],
        out_specs=pl.BlockSpec(
            (ROWS, D_OUT), lambda i: (jnp.where(i < HALF, 0, i - HALF), 0)),
        out_shape=jax.ShapeDtypeStruct((N, D_OUT), jnp.float32),
        scratch_shapes=[
            pltpu.VMEM((N, D_IN), jnp.float32),
            pltpu.VMEM((N, D_HID), jnp.float32),
            pltpu.VMEM((N, D_HID), jnp.float32),
            pltpu.VMEM((N, D_OUT), jnp.bfloat16),
            pltpu.VMEM((CB, ROWS, N), jnp.bfloat16),
            pltpu.SemaphoreType.DMA,
        ],
        compiler_params=pltpu.CompilerParams(
            dimension_semantics=("arbitrary",),
            vmem_limit_bytes=63 * 1024 * 1024,
        ),
    )(adj, x, W1, b1.reshape(1, D_HID), scale, W2, b2.reshape(1, D_OUT))
    return out


# speedup vs baseline: 1.1315x; 1.0054x over previous
import numpy as np
import jax
import jax.numpy as jnp
from jax.experimental import pallas as pl
from jax.experimental.pallas import tpu as pltpu

N = 10000
D_IN = 128
D_HID = 64
D_OUT = 40
P_DROP = 0.5
ROWS = 400
HALF = N // ROWS
CB = 2
UNC = HALF - CB


def _make_scale():
    keep = jax.random.bernoulli(jax.random.key(42), 1.0 - P_DROP, (N, D_HID))
    return jnp.where(keep, 1.0 / (1.0 - P_DROP), 0.0).astype(jnp.bfloat16)


try:
    with jax.default_device(jax.local_devices(backend="cpu")[0]):
        _SCALE = np.asarray(jax.jit(_make_scale)())
except Exception:
    _SCALE = np.asarray(_make_scale())


def _adj_index(i):
    j = i - HALF
    return (jnp.where(i < HALF, i, jnp.minimum(j, UNC - 1)), 0)


def _body(adj_ref, x_hbm, w1_ref, b1_ref, scale_ref, w2_ref, b2_ref,
          out_ref, x_vmem, s1, mid, s2b, cache, sem):
    i = pl.program_id(0)

    @pl.when(i == 0)
    def _():
        cp = pltpu.make_async_copy(x_hbm, x_vmem, sem)
        cp.start()
        cp.wait()
        s1[:] = jnp.dot(x_vmem[:], w1_ref[:],
                        preferred_element_type=jnp.float32)

    @pl.when(i < HALF)
    def _():
        m = jnp.dot(adj_ref[:], s1[:], preferred_element_type=jnp.float32)
        mid[pl.ds(i * ROWS, ROWS), :] = jnp.maximum(
            (m + b1_ref[:]) * scale_ref[:].astype(jnp.float32),
            0.0).astype(jnp.bfloat16)

        @pl.when(i >= UNC)
        def _():
            cache[i - UNC] = adj_ref[:].astype(jnp.bfloat16)

    @pl.when(i == HALF)
    def _():
        s2 = jnp.dot(mid[:], w2_ref[:].astype(jnp.bfloat16),
                     preferred_element_type=jnp.float32)
        s1[:, :D_OUT] = s2
        s2b[:] = s2.astype(jnp.bfloat16)

    def _finish(o):
        o = o + b2_ref[:]
        o = o - jnp.max(o, axis=1, keepdims=True)
        out_ref[:] = o - jnp.log(jnp.sum(jnp.exp(o), axis=1, keepdims=True))

    @pl.when((i >= HALF) & (i < HALF + UNC))
    def _():
        _finish(jnp.dot(adj_ref[:], s1[:, :D_OUT],
                        preferred_element_type=jnp.float32))

    @pl.when(i >= HALF + UNC)
    def _():
        _finish(jnp.dot(cache[i - HALF - UNC], s2b[:],
                        preferred_element_type=jnp.float32))


def kernel(input, adj, W1, b1, W2, b2):
    x = input.astype(jnp.float32)
    scale = jnp.asarray(_SCALE)

    out = pl.pallas_call(
        _body,
        grid=(2 * HALF,),
        in_specs=[
            pl.BlockSpec((ROWS, N), _adj_index),
            pl.BlockSpec(memory_space=pl.ANY),
            pl.BlockSpec((D_IN, D_HID), lambda i: (0, 0)),
            pl.BlockSpec((1, D_HID), lambda i: (0, 0)),
            pl.BlockSpec((ROWS, D_HID),
                         lambda i: (jnp.where(i < HALF, i, HALF - 1), 0)),
            pl.BlockSpec((D_HID, D_OUT), lambda i: (0, 0)),
            pl.BlockSpec((1, D_OUT), lambda i: (0, 0)),
        ],
        out_specs=pl.BlockSpec(
            (ROWS, D_OUT), lambda i: (jnp.where(i < HALF, 0, i - HALF), 0)),
        out_shape=jax.ShapeDtypeStruct((N, D_OUT), jnp.float32),
        scratch_shapes=[
            pltpu.VMEM((N, D_IN), jnp.float32),
            pltpu.VMEM((N, D_HID), jnp.float32),
            pltpu.VMEM((N, D_HID), jnp.bfloat16),
            pltpu.VMEM((N, D_OUT), jnp.bfloat16),
            pltpu.VMEM((CB, ROWS, N), jnp.bfloat16),
            pltpu.SemaphoreType.DMA,
        ],
        compiler_params=pltpu.CompilerParams(
            dimension_semantics=("arbitrary",),
            vmem_limit_bytes=63 * 1024 * 1024,
        ),
    )(adj, x, W1, b1.reshape(1, D_HID), scale, W2, b2.reshape(1, D_OUT))
    return out


# speedup vs baseline: 1.1409x; 1.0082x over previous
import numpy as np
import jax
import jax.numpy as jnp
from jax.experimental import pallas as pl
from jax.experimental.pallas import tpu as pltpu

N = 10000
D_IN = 128
D_HID = 64
D_OUT = 40
P_DROP = 0.5
ROWS = 400
HALF = N // ROWS
CB = 2
UNC = HALF - CB - 1


def _make_scale():
    keep = jax.random.bernoulli(jax.random.key(42), 1.0 - P_DROP, (N, D_HID))
    return jnp.where(keep, 1.0 / (1.0 - P_DROP), 0.0).astype(jnp.bfloat16)


try:
    with jax.default_device(jax.local_devices(backend="cpu")[0]):
        _SCALE = np.asarray(jax.jit(_make_scale)())
except Exception:
    _SCALE = np.asarray(_make_scale())


def _adj_index(i):
    return (jnp.where(i < HALF, i,
                      jnp.where(i == HALF, HALF - 1,
                                jnp.minimum(i - HALF - 1, UNC - 1))), 0)


def _body(adj_ref, x_hbm, w1_ref, b1_ref, scale_ref, w2_ref, b2_ref,
          out_ref, x_vmem, s1, mid, s2b, cache, sem):
    i = pl.program_id(0)

    @pl.when(i == 0)
    def _():
        cp = pltpu.make_async_copy(x_hbm, x_vmem, sem)
        cp.start()
        cp.wait()
        s1[:] = jnp.dot(x_vmem[:], w1_ref[:],
                        preferred_element_type=jnp.float32)

    @pl.when(i < HALF)
    def _():
        m = jnp.dot(adj_ref[:], s1[:], preferred_element_type=jnp.float32)
        mid[pl.ds(i * ROWS, ROWS), :] = jnp.maximum(
            (m + b1_ref[:]) * scale_ref[:].astype(jnp.float32),
            0.0).astype(jnp.bfloat16)

        @pl.when((i >= UNC) & (i < HALF - 1))
        def _():
            cache[i - UNC] = adj_ref[:].astype(jnp.bfloat16)

    @pl.when(i == HALF)
    def _():
        s2 = jnp.dot(mid[:], w2_ref[:].astype(jnp.bfloat16),
                     preferred_element_type=jnp.float32)
        s1[:, :D_OUT] = s2
        s2b[:] = s2.astype(jnp.bfloat16)

    def _finish(o):
        o = o + b2_ref[:]
        o = o - jnp.max(o, axis=1, keepdims=True)
        out_ref[:] = o - jnp.log(jnp.sum(jnp.exp(o), axis=1, keepdims=True))

    @pl.when((i >= HALF) & (i < HALF + UNC + 1))
    def _():
        _finish(jnp.dot(adj_ref[:], s1[:, :D_OUT],
                        preferred_element_type=jnp.float32))

    @pl.when(i >= HALF + UNC + 1)
    def _():
        _finish(jnp.dot(cache[i - HALF - UNC - 1], s2b[:],
                        preferred_element_type=jnp.float32))


def kernel(input, adj, W1, b1, W2, b2):
    x = input.astype(jnp.float32)
    scale = jnp.asarray(_SCALE)

    out = pl.pallas_call(
        _body,
        grid=(2 * HALF,),
        in_specs=[
            pl.BlockSpec((ROWS, N), _adj_index),
            pl.BlockSpec(memory_space=pl.ANY),
            pl.BlockSpec((D_IN, D_HID), lambda i: (0, 0)),
            pl.BlockSpec((1, D_HID), lambda i: (0, 0)),
            pl.BlockSpec((ROWS, D_HID),
                         lambda i: (jnp.where(i < HALF, i, HALF - 1), 0)),
            pl.BlockSpec((D_HID, D_OUT), lambda i: (0, 0)),
            pl.BlockSpec((1, D_OUT), lambda i: (0, 0)),
        ],
        out_specs=pl.BlockSpec(
            (ROWS, D_OUT),
            lambda i: (jnp.where(i <= HALF, HALF - 1, i - HALF - 1), 0)),
        out_shape=jax.ShapeDtypeStruct((N, D_OUT), jnp.float32),
        scratch_shapes=[
            pltpu.VMEM((N, D_IN), jnp.float32),
            pltpu.VMEM((N, D_HID), jnp.float32),
            pltpu.VMEM((N, D_HID), jnp.bfloat16),
            pltpu.VMEM((N, D_OUT), jnp.bfloat16),
            pltpu.VMEM((CB, ROWS, N), jnp.bfloat16),
            pltpu.SemaphoreType.DMA,
        ],
        compiler_params=pltpu.CompilerParams(
            dimension_semantics=("arbitrary",),
            vmem_limit_bytes=63 * 1024 * 1024,
        ),
    )(adj, x, W1, b1.reshape(1, D_HID), scale, W2, b2.reshape(1, D_OUT))
    return out
